# initial kernel scaffold (unmeasured)
import jax
import jax.numpy as jnp
from jax import lax
from jax.experimental import pallas as pl
from jax.experimental.pallas import tpu as pltpu

N_DEV = 4


def kernel(x, w_mat, scale_x, scale_w):
    m_per, k = x.shape
    n_total = w_mat.shape[1]
    n_per = n_total // N_DEV
    m_half = m_per // 2

    my_pos = lax.axis_index("i")
    w_my = lax.dynamic_slice_in_dim(w_mat, my_pos * n_per, n_per, axis=1)

    def body(x_ref, w_ref, sx_ref, sw_ref, out_ref, rbuf, hbuf,
             ssems, rsems):
        my = lax.axis_index("i")
        left = (my - 1) % N_DEV
        right = (my + 1) % N_DEV
        scale = sx_ref[0] * sw_ref[0]

        barrier_sem = pltpu.get_barrier_semaphore()
        for nbr in (left, right):
            pl.semaphore_signal(
                barrier_sem, inc=1,
                device_id=(nbr,), device_id_type=pl.DeviceIdType.MESH,
            )
        pl.semaphore_wait(barrier_sem, 2)

        def gemm(chunk_fp8, row_start, rows):
            acc = jnp.dot(chunk_fp8, w_ref[...],
                          preferred_element_type=jnp.float32)
            out_ref[pl.ds(row_start, rows), :] = acc * scale

        p1r = pltpu.make_async_remote_copy(
            src_ref=x_ref, dst_ref=rbuf.at[0],
            send_sem=ssems.at[0], recv_sem=rsems.at[0],
            device_id=(right,), device_id_type=pl.DeviceIdType.MESH,
        )
        p1l = pltpu.make_async_remote_copy(
            src_ref=x_ref, dst_ref=rbuf.at[1],
            send_sem=ssems.at[1], recv_sem=rsems.at[1],
            device_id=(left,), device_id_type=pl.DeviceIdType.MESH,
        )
        p1r.start()
        p1l.start()

        gemm(x_ref[...], my * m_per, m_per)

        p1r.wait_recv()
        p2r = pltpu.make_async_remote_copy(
            src_ref=rbuf.at[0, pl.ds(0, m_half)], dst_ref=hbuf.at[0],
            send_sem=ssems.at[2], recv_sem=rsems.at[2],
            device_id=(right,), device_id_type=pl.DeviceIdType.MESH,
        )
        p2r.start()
        gemm(rbuf[0], left * m_per, m_per)

        p1l.wait_recv()
        p2l = pltpu.make_async_remote_copy(
            src_ref=rbuf.at[1, pl.ds(m_half, m_half)], dst_ref=hbuf.at[1],
            send_sem=ssems.at[3], recv_sem=rsems.at[3],
            device_id=(left,), device_id_type=pl.DeviceIdType.MESH,
        )
        p2l.start()
        gemm(rbuf[1], right * m_per, m_per)

        opp = (my + 2) % N_DEV
        p2r.wait_recv()
        gemm(hbuf[0], opp * m_per, m_half)
        p2l.wait_recv()
        gemm(hbuf[1], opp * m_per + m_half, m_half)

        p1r.wait_send()
        p1l.wait_send()
        p2r.wait_send()
        p2l.wait_send()

    return pl.pallas_call(
        body,
        out_shape=jax.ShapeDtypeStruct((N_DEV * m_per, n_per), jnp.float32),
        in_specs=[
            pl.BlockSpec(memory_space=pltpu.VMEM),
            pl.BlockSpec(memory_space=pltpu.VMEM),
            pl.BlockSpec(memory_space=pltpu.SMEM),
            pl.BlockSpec(memory_space=pltpu.SMEM),
        ],
        out_specs=pl.BlockSpec(memory_space=pltpu.VMEM),
        scratch_shapes=[
            pltpu.VMEM((2, m_per, k), x.dtype),
            pltpu.VMEM((2, m_half, k), x.dtype),
            pltpu.SemaphoreType.DMA((4,)),
            pltpu.SemaphoreType.DMA((4,)),
        ],
        compiler_params=pltpu.CompilerParams(collective_id=0),
    )(x, w_my, scale_x, scale_w)


# baseline (device time: 152955 ns/iter reference)
import jax
import jax.numpy as jnp
from jax import lax
from jax.experimental import pallas as pl
from jax.experimental.pallas import tpu as pltpu

N_DEV = 4


def kernel(x, w_mat, scale_x, scale_w):
    m_per, k = x.shape
    n_total = w_mat.shape[1]
    n_per = n_total // N_DEV
    m_half = m_per // 2

    my_pos = lax.axis_index("i")
    w_my = lax.dynamic_slice_in_dim(w_mat, my_pos * n_per, n_per, axis=1)
    x = x.astype(jnp.float8_e5m2)
    w_my = w_my.astype(jnp.float8_e5m2)

    def body(x_ref, w_ref, sx_ref, sw_ref, out_ref, rbuf, hbuf,
             ssems, rsems):
        my = lax.axis_index("i")
        left = (my - 1) % N_DEV
        right = (my + 1) % N_DEV
        scale = sx_ref[0] * sw_ref[0]

        barrier_sem = pltpu.get_barrier_semaphore()
        for nbr in (left, right):
            pl.semaphore_signal(
                barrier_sem, inc=1,
                device_id=(nbr,), device_id_type=pl.DeviceIdType.MESH,
            )
        pl.semaphore_wait(barrier_sem, 2)

        def gemm(chunk_ref, row_start, rows):
            for r0 in range(0, rows, m_half):
                acc = jnp.dot(chunk_ref[pl.ds(r0, m_half)], w_ref[...],
                              preferred_element_type=jnp.float32)
                out_ref[pl.ds(row_start + r0, m_half), :] = acc * scale

        p1r = pltpu.make_async_remote_copy(
            src_ref=x_ref, dst_ref=rbuf.at[0],
            send_sem=ssems.at[0], recv_sem=rsems.at[0],
            device_id=(right,), device_id_type=pl.DeviceIdType.MESH,
        )
        p1l = pltpu.make_async_remote_copy(
            src_ref=x_ref, dst_ref=rbuf.at[1],
            send_sem=ssems.at[1], recv_sem=rsems.at[1],
            device_id=(left,), device_id_type=pl.DeviceIdType.MESH,
        )
        p1r.start()
        p1l.start()

        gemm(x_ref, my * m_per, m_per)

        p1r.wait_recv()
        p2r = pltpu.make_async_remote_copy(
            src_ref=rbuf.at[0, pl.ds(0, m_half)], dst_ref=hbuf.at[0],
            send_sem=ssems.at[2], recv_sem=rsems.at[2],
            device_id=(right,), device_id_type=pl.DeviceIdType.MESH,
        )
        p2r.start()
        gemm(rbuf.at[0], left * m_per, m_per)

        p1l.wait_recv()
        p2l = pltpu.make_async_remote_copy(
            src_ref=rbuf.at[1, pl.ds(m_half, m_half)], dst_ref=hbuf.at[1],
            send_sem=ssems.at[3], recv_sem=rsems.at[3],
            device_id=(left,), device_id_type=pl.DeviceIdType.MESH,
        )
        p2l.start()
        gemm(rbuf.at[1], right * m_per, m_per)

        opp = (my + 2) % N_DEV
        p2r.wait_recv()
        gemm(hbuf.at[0], opp * m_per, m_half)
        p2l.wait_recv()
        gemm(hbuf.at[1], opp * m_per + m_half, m_half)

        p1r.wait_send()
        p1l.wait_send()
        p2r.wait_send()
        p2l.wait_send()

    return pl.pallas_call(
        body,
        out_shape=jax.ShapeDtypeStruct((N_DEV * m_per, n_per), jnp.float32),
        in_specs=[
            pl.BlockSpec(memory_space=pltpu.VMEM),
            pl.BlockSpec(memory_space=pltpu.VMEM),
            pl.BlockSpec(memory_space=pltpu.SMEM),
            pl.BlockSpec(memory_space=pltpu.SMEM),
        ],
        out_specs=pl.BlockSpec(memory_space=pltpu.VMEM),
        scratch_shapes=[
            pltpu.VMEM((2, m_per, k), x.dtype),
            pltpu.VMEM((2, m_half, k), x.dtype),
            pltpu.SemaphoreType.DMA((4,)),
            pltpu.SemaphoreType.DMA((4,)),
        ],
        compiler_params=pltpu.CompilerParams(
            collective_id=0,
            vmem_limit_bytes=64 * 1024 * 1024,
        ),
    )(x, w_my, scale_x, scale_w)


# device time: 146155 ns/iter; 1.0465x vs baseline; 1.0465x over previous
import jax
import jax.numpy as jnp
from jax import lax
from jax.experimental import pallas as pl
from jax.experimental.pallas import tpu as pltpu

N_DEV = 4


def kernel(x, w_mat, scale_x, scale_w):
    m_per, k = x.shape
    n_total = w_mat.shape[1]
    n_per = n_total // N_DEV
    m_half = m_per // 2

    my_pos = lax.axis_index("i")
    w_my = lax.dynamic_slice_in_dim(w_mat, my_pos * n_per, n_per, axis=1)
    x = x.astype(jnp.float8_e5m2)
    w_my = w_my.astype(jnp.float8_e5m2)

    def body(x_ref, w_ref, sx_ref, sw_ref, out_ref, rbuf, hbuf,
             ssems, rsems):
        my = lax.axis_index("i")
        left = (my - 1) % N_DEV
        right = (my + 1) % N_DEV
        scale = sx_ref[0] * sw_ref[0]

        barrier_sem = pltpu.get_barrier_semaphore()
        for nbr in (left, right):
            pl.semaphore_signal(
                barrier_sem, inc=1,
                device_id=(nbr,), device_id_type=pl.DeviceIdType.MESH,
            )
        pl.semaphore_wait(barrier_sem, 2)

        H = m_half
        Q = m_half // 2

        def gemm(src_ref, src_off, out_off, rows):
            acc = jnp.dot(src_ref[pl.ds(src_off, rows)], w_ref[...],
                          preferred_element_type=jnp.float32)
            out_ref[pl.ds(out_off, rows), :] = acc * scale

        def rc(src, dst, sem_i, dev):
            return pltpu.make_async_remote_copy(
                src_ref=src, dst_ref=dst,
                send_sem=ssems.at[sem_i], recv_sem=rsems.at[sem_i],
                device_id=(dev,), device_id_type=pl.DeviceIdType.MESH,
            )

        p1r = [rc(x_ref.at[pl.ds(h * H, H)], rbuf.at[0, pl.ds(h * H, H)],
                  h, right) for h in range(2)]
        p1l = [rc(x_ref.at[pl.ds(h * H, H)], rbuf.at[1, pl.ds(h * H, H)],
                  2 + h, left) for h in range(2)]
        p2r = [rc(rbuf.at[0, pl.ds(q * Q, Q)], hbuf.at[0, pl.ds(q * Q, Q)],
                  4 + q, right) for q in range(2)]
        p2l = [rc(rbuf.at[1, pl.ds(H + q * Q, Q)],
                  hbuf.at[1, pl.ds(q * Q, Q)], 6 + q, left)
               for q in range(2)]

        for r in p1r + p1l:
            r.start()

        gemm(x_ref, 0, my * m_per, H)
        gemm(x_ref, H, my * m_per + H, H)

        p1r[0].wait_recv()
        p2r[0].start()
        p2r[1].start()
        gemm(rbuf.at[0], 0, left * m_per, H)

        p1l[0].wait_recv()
        gemm(rbuf.at[1], 0, right * m_per, H)

        p1r[1].wait_recv()
        gemm(rbuf.at[0], H, left * m_per + H, H)

        p1l[1].wait_recv()
        p2l[0].start()
        p2l[1].start()
        gemm(rbuf.at[1], H, right * m_per + H, H)

        opp = (my + 2) % N_DEV
        p2r[0].wait_recv()
        gemm(hbuf.at[0], 0, opp * m_per, Q)
        p2l[0].wait_recv()
        gemm(hbuf.at[1], 0, opp * m_per + H, Q)
        p2r[1].wait_recv()
        gemm(hbuf.at[0], Q, opp * m_per + Q, Q)
        p2l[1].wait_recv()
        gemm(hbuf.at[1], Q, opp * m_per + H + Q, Q)

        for r in p1r + p1l + p2r + p2l:
            r.wait_send()

    return pl.pallas_call(
        body,
        out_shape=jax.ShapeDtypeStruct((N_DEV * m_per, n_per), jnp.float32),
        in_specs=[
            pl.BlockSpec(memory_space=pltpu.VMEM),
            pl.BlockSpec(memory_space=pltpu.VMEM),
            pl.BlockSpec(memory_space=pltpu.SMEM),
            pl.BlockSpec(memory_space=pltpu.SMEM),
        ],
        out_specs=pl.BlockSpec(memory_space=pltpu.VMEM),
        scratch_shapes=[
            pltpu.VMEM((2, m_per, k), x.dtype),
            pltpu.VMEM((2, m_half, k), x.dtype),
            pltpu.SemaphoreType.DMA((8,)),
            pltpu.SemaphoreType.DMA((8,)),
        ],
        compiler_params=pltpu.CompilerParams(
            collective_id=0,
            vmem_limit_bytes=64 * 1024 * 1024,
        ),
    )(x, w_my, scale_x, scale_w)


# device time: 116028 ns/iter; 1.3183x vs baseline; 1.2597x over previous
import jax
import jax.numpy as jnp
from jax import lax
from jax.experimental import pallas as pl
from jax.experimental.pallas import tpu as pltpu

N_DEV = 4


def kernel(x, w_mat, scale_x, scale_w):
    m_per, k = x.shape
    n_total = w_mat.shape[1]
    n_per = n_total // N_DEV
    H = m_per // 2
    Q = H // 2
    KS = 1024
    n_strips = k // KS

    x = x.astype(jnp.float8_e5m2)

    def body(x_ref, w_ref, sx_ref, sw_ref, out_ref, rbuf, hbuf, w8,
             wstrip, stg, ssems, rsems, wsems, osems):
        my = lax.axis_index("i")
        left = (my - 1) % N_DEV
        right = (my + 1) % N_DEV
        opp = (my + 2) % N_DEV
        scale = sx_ref[0] * sw_ref[0]

        def w_dma(j):
            d = pltpu.make_async_copy(
                w_ref.at[pl.ds(j * KS, KS), pl.ds(my * n_per, n_per)],
                wstrip.at[j % 2],
                wsems.at[j % 2],
            )
            d.start()
            return d

        wd = {j: w_dma(j) for j in range(2)}

        barrier_sem = pltpu.get_barrier_semaphore()
        for nbr in (left, right):
            pl.semaphore_signal(
                barrier_sem, inc=1,
                device_id=(nbr,), device_id_type=pl.DeviceIdType.MESH,
            )
        pl.semaphore_wait(barrier_sem, 2)

        def rc(src, dst, sem_i, dev):
            return pltpu.make_async_remote_copy(
                src_ref=src, dst_ref=dst,
                send_sem=ssems.at[sem_i], recv_sem=rsems.at[sem_i],
                device_id=(dev,), device_id_type=pl.DeviceIdType.MESH,
            )

        p1r = [rc(x_ref.at[pl.ds(h * H, H)], rbuf.at[0, pl.ds(h * H, H)],
                  h, right) for h in range(2)]
        p1l = [rc(x_ref.at[pl.ds(h * H, H)], rbuf.at[1, pl.ds(h * H, H)],
                  2 + h, left) for h in range(2)]
        p2r = [rc(rbuf.at[0, pl.ds(q * Q, Q)], hbuf.at[0, pl.ds(q * Q, Q)],
                  4 + q, right) for q in range(2)]
        p2l = [rc(rbuf.at[1, pl.ds(H + q * Q, Q)],
                  hbuf.at[1, pl.ds(q * Q, Q)], 6 + q, left)
               for q in range(2)]

        for r in p1r + p1l:
            r.start()

        for j in range(n_strips):
            wd[j].wait()
            w8[pl.ds(j * KS, KS), :] = wstrip[j % 2].astype(jnp.float8_e5m2)
            if j + 2 < n_strips:
                wd[j + 2] = w_dma(j + 2)

        out_dmas = []

        def gemm(src_ref, src_off, out_off, rows):
            i = len(out_dmas)
            slot = i % 2
            if i >= 2:
                out_dmas[i - 2].wait()
            acc = jnp.dot(src_ref[pl.ds(src_off, rows)], w8[...],
                          preferred_element_type=jnp.float32)
            stg[slot, pl.ds(0, rows), :] = acc * scale
            d = pltpu.make_async_copy(
                stg.at[slot, pl.ds(0, rows)],
                out_ref.at[pl.ds(out_off, rows)],
                osems.at[slot],
            )
            d.start()
            out_dmas.append(d)

        gemm(x_ref, 0, my * m_per, H)
        gemm(x_ref, H, my * m_per + H, H)

        p1r[0].wait_recv()
        p2r[0].start()
        p2r[1].start()
        gemm(rbuf.at[0], 0, left * m_per, H)

        p1l[0].wait_recv()
        gemm(rbuf.at[1], 0, right * m_per, H)

        p1r[1].wait_recv()
        gemm(rbuf.at[0], H, left * m_per + H, H)

        p1l[1].wait_recv()
        p2l[0].start()
        p2l[1].start()
        gemm(rbuf.at[1], H, right * m_per + H, H)

        p2r[0].wait_recv()
        gemm(hbuf.at[0], 0, opp * m_per, Q)
        p2l[0].wait_recv()
        gemm(hbuf.at[1], 0, opp * m_per + H, Q)
        p2r[1].wait_recv()
        gemm(hbuf.at[0], Q, opp * m_per + Q, Q)
        p2l[1].wait_recv()
        gemm(hbuf.at[1], Q, opp * m_per + H + Q, Q)

        out_dmas[-2].wait()
        out_dmas[-1].wait()
        for r in p1r + p1l + p2r + p2l:
            r.wait_send()

    return pl.pallas_call(
        body,
        out_shape=jax.ShapeDtypeStruct((N_DEV * m_per, n_per), jnp.float32),
        in_specs=[
            pl.BlockSpec(memory_space=pltpu.VMEM),
            pl.BlockSpec(memory_space=pl.ANY),
            pl.BlockSpec(memory_space=pltpu.SMEM),
            pl.BlockSpec(memory_space=pltpu.SMEM),
        ],
        out_specs=pl.BlockSpec(memory_space=pl.ANY),
        scratch_shapes=[
            pltpu.VMEM((2, m_per, k), x.dtype),
            pltpu.VMEM((2, H, k), x.dtype),
            pltpu.VMEM((k, n_per), x.dtype),
            pltpu.VMEM((2, KS, n_per), jnp.float32),
            pltpu.VMEM((2, H, n_per), jnp.float32),
            pltpu.SemaphoreType.DMA((8,)),
            pltpu.SemaphoreType.DMA((8,)),
            pltpu.SemaphoreType.DMA((2,)),
            pltpu.SemaphoreType.DMA((2,)),
        ],
        compiler_params=pltpu.CompilerParams(
            collective_id=0,
            vmem_limit_bytes=64 * 1024 * 1024,
        ),
    )(x, w_mat, scale_x, scale_w)
